# Initial kernel scaffold; baseline (speedup 1.0000x reference)
import jax
import jax.numpy as jnp
from jax.experimental import pallas as pl


def kernel(x, table):
    raise NotImplementedError("write your pallas kernel here")


# speedup vs baseline: 5.9269x; 5.9269x over previous
import functools

import jax
import jax.numpy as jnp
from jax import lax
from jax.experimental import pallas as pl
from jax.experimental.pallas import tpu as pltpu
from jax.experimental.pallas import tpu_sc as plsc

_NC, _NS = 2, 16
_NW = _NC * _NS
_D = 128
_CHUNK = 128


@functools.partial(jax.jit, static_argnums=(2,))
def _gather_rows(flat_idx, table, B):
    b_per_w = B // _NW
    n_chunks = b_per_w // _CHUNK

    mesh = plsc.VectorSubcoreMesh(
        core_axis_name="c", subcore_axis_name="s",
        num_cores=_NC, num_subcores=_NS)

    @functools.partial(
        pl.kernel,
        out_type=jax.ShapeDtypeStruct((B, _D), jnp.float32),
        mesh=mesh,
        scratch_types=[
            pltpu.VMEM((_CHUNK,), jnp.int32),
            pltpu.VMEM((_CHUNK, _D), jnp.float32),
            pltpu.SemaphoreType.DMA,
        ],
    )
    def k(idx_hbm, table_hbm, out_hbm, idx_v, rows_v, sem):
        wid = lax.axis_index("s") * _NC + lax.axis_index("c")
        base = wid * b_per_w

        def body(i, _):
            off = base + i * _CHUNK
            pltpu.sync_copy(idx_hbm.at[pl.ds(off, _CHUNK)], idx_v)
            pltpu.async_copy(table_hbm.at[idx_v], rows_v, sem).wait()
            pltpu.sync_copy(rows_v, out_hbm.at[pl.ds(off, _CHUNK)])
            return ()

        lax.fori_loop(0, n_chunks, body, ())

    return k(flat_idx, table)


def kernel(x, table):
    B0, S = x.shape
    B = B0 * S
    out = _gather_rows(x.reshape(B), table, B)
    return out.reshape(B0, S, _D)


# speedup vs baseline: 10.8669x; 1.8335x over previous
import functools

import jax
import jax.numpy as jnp
from jax import lax
from jax.experimental import pallas as pl
from jax.experimental.pallas import tpu as pltpu
from jax.experimental.pallas import tpu_sc as plsc

_NC, _NS = 2, 16
_NW = _NC * _NS
_D = 128
_CH = 128
_NBUF = 4


@functools.partial(jax.jit, static_argnums=(2,))
def _gather_rows(flat_idx, table, B):
    b_per_w = B // _NW
    n_chunks = b_per_w // _CH
    assert n_chunks >= 8 and (n_chunks - 4) % _NBUF == 0

    mesh = plsc.VectorSubcoreMesh(
        core_axis_name="c", subcore_axis_name="s",
        num_cores=_NC, num_subcores=_NS)

    @functools.partial(
        pl.kernel,
        out_type=jax.ShapeDtypeStruct((B, _D), jnp.float32),
        mesh=mesh,
        scratch_types=[
            pltpu.VMEM((_NBUF, _CH), jnp.int32),
            pltpu.VMEM((_NBUF, _CH, _D), jnp.float32),
            [pltpu.SemaphoreType.DMA] * _NBUF,
            [pltpu.SemaphoreType.DMA] * _NBUF,
        ],
    )
    def k(idx_hbm, table_hbm, out_hbm, idx_v, rows_v, gsem, ssem):
        wid = lax.axis_index("s") * _NC + lax.axis_index("c")
        base = wid * b_per_w

        def issue_gather(i, b):
            pltpu.sync_copy(idx_hbm.at[pl.ds(base + i * _CH, _CH)], idx_v.at[b])
            pltpu.async_copy(table_hbm.at[idx_v.at[b]], rows_v.at[b], gsem[b])

        def wait_gather(b):
            pltpu.make_async_copy(
                table_hbm.at[idx_v.at[b]], rows_v.at[b], gsem[b]).wait()

        def issue_store(i, b):
            pltpu.async_copy(
                rows_v.at[b], out_hbm.at[pl.ds(base + i * _CH, _CH)], ssem[b])

        def wait_store(i, b):
            pltpu.make_async_copy(
                rows_v.at[b], out_hbm.at[pl.ds(base + i * _CH, _CH)],
                ssem[b]).wait()

        issue_gather(0, 0)
        issue_gather(1, 1)
        issue_gather(2, 2)
        wait_gather(0)
        issue_store(0, 0)
        issue_gather(3, 3)
        wait_gather(1)
        issue_store(1, 1)

        def outer(g, _):
            for u in range(_NBUF):
                i = 2 + g * _NBUF + u
                b = (2 + u) % _NBUF
                pb = u
                wait_store(i - 2, pb)
                issue_gather(i + 2, pb)
                wait_gather(b)
                issue_store(i, b)
            return ()

        lax.fori_loop(0, (n_chunks - 4) // _NBUF, outer, ())

        n = n_chunks
        wait_store(n - 4, (n - 4) % _NBUF)
        wait_gather((n - 2) % _NBUF)
        issue_store(n - 2, (n - 2) % _NBUF)
        wait_store(n - 3, (n - 3) % _NBUF)
        wait_gather((n - 1) % _NBUF)
        issue_store(n - 1, (n - 1) % _NBUF)
        wait_store(n - 2, (n - 2) % _NBUF)
        wait_store(n - 1, (n - 1) % _NBUF)

    return k(flat_idx, table)


def kernel(x, table):
    B0, S = x.shape
    B = B0 * S
    out = _gather_rows(x.reshape(B), table, B)
    return out.reshape(B0, S, _D)


# speedup vs baseline: 10.8700x; 1.0003x over previous
import functools

import jax
import jax.numpy as jnp
from jax import lax
from jax.experimental import pallas as pl
from jax.experimental.pallas import tpu as pltpu
from jax.experimental.pallas import tpu_sc as plsc

_NC, _NS = 2, 16
_NW = _NC * _NS
_D = 128
_CH = 128
_NBUF = 4


@functools.partial(jax.jit, static_argnums=(2,))
def _gather_rows(flat_idx, table, B):
    b_per_w = B // _NW
    n = b_per_w // _CH
    assert n >= 12 and n % _NBUF == 0

    mesh = plsc.VectorSubcoreMesh(
        core_axis_name="c", subcore_axis_name="s",
        num_cores=_NC, num_subcores=_NS)

    @functools.partial(
        pl.kernel,
        out_type=jax.ShapeDtypeStruct((B, _D), jnp.float32),
        mesh=mesh,
        scratch_types=[
            pltpu.VMEM((_NBUF, _CH), jnp.int32),
            pltpu.VMEM((_NBUF, _CH, _D), jnp.float32),
            [pltpu.SemaphoreType.DMA] * _NBUF,
            [pltpu.SemaphoreType.DMA] * _NBUF,
            [pltpu.SemaphoreType.DMA] * _NBUF,
        ],
    )
    def k(idx_hbm, table_hbm, out_hbm, idx_v, rows_v, isem, gsem, ssem):
        wid = lax.axis_index("s") * _NC + lax.axis_index("c")
        base = wid * b_per_w

        def issue_idx(i, b):
            pltpu.async_copy(
                idx_hbm.at[pl.ds(base + i * _CH, _CH)], idx_v.at[b], isem[b])

        def wait_idx(i, b):
            pltpu.make_async_copy(
                idx_hbm.at[pl.ds(base + i * _CH, _CH)], idx_v.at[b],
                isem[b]).wait()

        def issue_gather(b):
            pltpu.async_copy(table_hbm.at[idx_v.at[b]], rows_v.at[b], gsem[b])

        def wait_gather(b):
            pltpu.make_async_copy(
                table_hbm.at[idx_v.at[b]], rows_v.at[b], gsem[b]).wait()

        def issue_store(i, b):
            pltpu.async_copy(
                rows_v.at[b], out_hbm.at[pl.ds(base + i * _CH, _CH)], ssem[b])

        def wait_store(i, b):
            pltpu.make_async_copy(
                rows_v.at[b], out_hbm.at[pl.ds(base + i * _CH, _CH)],
                ssem[b]).wait()

        def slot(i, u, w_store=True, p_gather=True, p_idx=True):
            pu = (u + 2) % _NBUF
            if w_store:
                wait_store(i - 2, pu)
            if p_gather:
                wait_idx(i + 2, pu)
                issue_gather(pu)
            wait_gather(u)
            if p_idx:
                issue_idx(i + 4, u)
            issue_store(i, u)

        for c in range(_NBUF):
            issue_idx(c, c)
        for c in range(2):
            wait_idx(c, c)
            issue_gather(c)
        slot(0, 0, w_store=False)
        slot(1, 1, w_store=False)

        def outer(g, _):
            for v in range(_NBUF):
                i = 2 + g * _NBUF + v
                slot(i, (2 + v) % _NBUF)
            return ()

        lax.fori_loop(0, (n - 8) // _NBUF, outer, ())

        slot(n - 6, (n - 6) % _NBUF)
        slot(n - 5, (n - 5) % _NBUF)
        slot(n - 4, (n - 4) % _NBUF, p_idx=False)
        slot(n - 3, (n - 3) % _NBUF, p_idx=False)
        slot(n - 2, (n - 2) % _NBUF, p_gather=False, p_idx=False)
        slot(n - 1, (n - 1) % _NBUF, p_gather=False, p_idx=False)
        wait_store(n - 2, (n - 2) % _NBUF)
        wait_store(n - 1, (n - 1) % _NBUF)

    return k(flat_idx, table)


def kernel(x, table):
    B0, S = x.shape
    B = B0 * S
    out = _gather_rows(x.reshape(B), table, B)
    return out.reshape(B0, S, _D)


# speedup vs baseline: 10.8811x; 1.0010x over previous
import functools

import jax
import jax.numpy as jnp
from jax import lax
from jax.experimental import pallas as pl
from jax.experimental.pallas import tpu as pltpu
from jax.experimental.pallas import tpu_sc as plsc

_NC, _NS = 2, 16
_NW = _NC * _NS
_D = 128
_CH = 256
_NBUF = 3
_GD = _NBUF - 2
_NSPLIT = _CH // 128


@functools.partial(jax.jit, static_argnums=(2,))
def _gather_rows(flat_idx, table, B):
    b_per_w = B // _NW
    n = b_per_w // _CH
    assert n * _CH == b_per_w and n >= 10
    steady = ((n - 8) // _NBUF) * _NBUF

    mesh = plsc.VectorSubcoreMesh(
        core_axis_name="c", subcore_axis_name="s",
        num_cores=_NC, num_subcores=_NS)

    @functools.partial(
        pl.kernel,
        out_type=jax.ShapeDtypeStruct((B, _D), jnp.float32),
        mesh=mesh,
        scratch_types=[
            pltpu.VMEM((_NBUF, _NSPLIT, 128), jnp.int32),
            pltpu.VMEM((_NBUF, _CH, _D), jnp.float32),
            [pltpu.SemaphoreType.DMA] * _NBUF,
            [pltpu.SemaphoreType.DMA] * _NBUF,
            [pltpu.SemaphoreType.DMA] * _NBUF,
        ],
    )
    def k(idx_hbm, table_hbm, out_hbm, idx_v, rows_v, isem, gsem, ssem):
        wid = lax.axis_index("s") * _NC + lax.axis_index("c")
        base = wid * b_per_w

        def issue_idx(i, b):
            for j in range(_NSPLIT):
                pltpu.async_copy(
                    idx_hbm.at[pl.ds(base + i * _CH + j * 128, 128)],
                    idx_v.at[b].at[j], isem[b])

        def wait_idx(i, b):
            for j in range(_NSPLIT):
                pltpu.make_async_copy(
                    idx_hbm.at[pl.ds(base + i * _CH + j * 128, 128)],
                    idx_v.at[b].at[j], isem[b]).wait()

        def issue_gather(b):
            for j in range(_NSPLIT):
                pltpu.async_copy(
                    table_hbm.at[idx_v.at[b].at[j]],
                    rows_v.at[b].at[pl.ds(j * 128, 128)], gsem[b])

        def wait_gather(b):
            for j in range(_NSPLIT):
                pltpu.make_async_copy(
                    table_hbm.at[idx_v.at[b].at[j]],
                    rows_v.at[b].at[pl.ds(j * 128, 128)], gsem[b]).wait()

        def issue_store(i, b):
            pltpu.async_copy(
                rows_v.at[b], out_hbm.at[pl.ds(base + i * _CH, _CH)], ssem[b])

        def wait_store(i, b):
            pltpu.make_async_copy(
                rows_v.at[b], out_hbm.at[pl.ds(base + i * _CH, _CH)],
                ssem[b]).wait()

        def slot(i, u, w_store=True, p_gather=True, p_idx=True):
            pu = (u + _GD) % _NBUF
            if w_store:
                wait_store(i - (_NBUF - _GD), pu)
            if p_gather:
                wait_idx(i + _GD, pu)
                issue_gather(pu)
            wait_gather(u)
            if p_idx:
                issue_idx(i + _NBUF, u)
            issue_store(i, u)

        for c in range(_NBUF):
            issue_idx(c, c)
        for c in range(_GD):
            wait_idx(c, c)
            issue_gather(c)
        slot(0, 0, w_store=False)
        slot(1, 1 % _NBUF, w_store=False)

        def outer(g, _):
            for v in range(_NBUF):
                i = 2 + g * _NBUF + v
                slot(i, (2 + v) % _NBUF)
            return ()

        lax.fori_loop(0, steady // _NBUF, outer, ())

        for c in range(2 + steady, n):
            slot(c, c % _NBUF,
                 p_gather=(c + _GD <= n - 1), p_idx=(c + _NBUF <= n - 1))
        wait_store(n - 2, (n - 2) % _NBUF)
        wait_store(n - 1, (n - 1) % _NBUF)

    return k(flat_idx, table)


def kernel(x, table):
    B0, S = x.shape
    B = B0 * S
    out = _gather_rows(x.reshape(B), table, B)
    return out.reshape(B0, S, _D)
